# baseline (speedup 1.0000x reference)
import functools

import jax
import jax.numpy as jnp
from jax import lax
from jax.experimental import pallas as pl
from jax.experimental.pallas import tpu as pltpu
from jax.experimental.pallas import tpu_sc as plsc

_N_FIELDS = 26
_VOCAB = 100000
_DIM = 16
_BATCH = 16384

_NC, _NS, _L = 2, 16, 16
_NW = _NC * _NS
_NSTRIPE = _N_FIELDS * _DIM
_SPW = _NSTRIPE // _NW

_C0 = 33408
_CHUNKS = (_C0, _C0, 33152)
_STARTS = (0, _C0, 2 * _C0)
_VTAIL = 2 * _C0 + 33152
_UNROLL = 8
_NIT = _BATCH // (_UNROLL * _L)

_mesh = plsc.VectorSubcoreMesh(
    core_axis_name="c", subcore_axis_name="s", num_cores=_NC, num_subcores=_NS
)


@functools.partial(
    pl.kernel,
    out_type=jax.ShapeDtypeStruct((_N_FIELDS, _DIM, _BATCH), jnp.float32),
    mesh=_mesh,
    compiler_params=pltpu.CompilerParams(
        needs_layout_passes=False, use_tc_tiling_on_sc=True
    ),
    scratch_types=[
        pltpu.VMEM((2 * _C0,), jnp.float32),
        pltpu.VMEM((_BATCH,), jnp.int32),
        pltpu.VMEM((_BATCH,), jnp.float32),
        pltpu.SemaphoreType.DMA,
        pltpu.SemaphoreType.DMA,
        pltpu.SemaphoreType.DMA,
        pltpu.SemaphoreType.DMA,
    ],
)
def _gather_kernel(
    xt_hbm, tab_hbm, tail_hbm, out_hbm, chunk_v, x_v, out_v, semc0, semc1, semo, semt
):
    wid = lax.axis_index("s") * _NC + lax.axis_index("c")
    s0 = wid * _SPW
    csems = (semc0, semc1)
    iota = lax.iota(jnp.int32, _L)

    def fd(i):
        s = s0 + i
        return s // _DIM, s % _DIM

    def fire_chunk(i, k):
        f, d = fd(i)
        slot = (3 * i + k) % 2
        pltpu.async_copy(
            tab_hbm.at[f, d, pl.ds(_STARTS[k], _CHUNKS[k])],
            chunk_v.at[pl.ds(slot * _C0, _CHUNKS[k])],
            csems[slot],
        )

    def wait_chunk(i, k):
        f, d = fd(i)
        slot = (3 * i + k) % 2
        pltpu.make_async_copy(
            tab_hbm.at[f, d, pl.ds(_STARTS[k], _CHUNKS[k])],
            chunk_v.at[pl.ds(slot * _C0, _CHUNKS[k])],
            csems[slot],
        ).wait()

    def fire_tail(i):
        f, d = fd(i)
        slot = (3 * i + 2) % 2
        pltpu.async_copy(
            tail_hbm.at[f, d],
            chunk_v.at[pl.ds(slot * _C0 + _CHUNKS[2], 128)],
            semt,
        )

    def wait_tail(i):
        f, d = fd(i)
        slot = (3 * i + 2) % 2
        pltpu.make_async_copy(
            tail_hbm.at[f, d],
            chunk_v.at[pl.ds(slot * _C0 + _CHUNKS[2], 128)],
            semt,
        ).wait()

    def load_x(f):
        pltpu.sync_copy(xt_hbm.at[f], x_v)

    def stripe_pass(i, k):
        cbase = ((3 * i + k) % 2) * _C0

        def grp(g, c):
            for j in range(_UNROLL):
                off = (g * _UNROLL + j) * _L
                xv = x_v[pl.ds(off, _L)]
                if k == 0:
                    idx = jnp.minimum(xv, _C0 - 1)
                    gathered = plsc.load_gather(chunk_v, [idx + cbase])
                    out_v[pl.ds(off, _L)] = gathered
                else:
                    t = xv - _STARTS[k]
                    if k == 1:
                        m = lax.convert_element_type(t, jnp.uint32) < _CHUNKS[1]
                    else:
                        m = t >= 0
                    gathered = plsc.load_gather(chunk_v, [t + cbase], mask=m)
                    plsc.store_scatter(out_v, [iota + off], gathered, mask=m)
            return c

        lax.fori_loop(0, _NIT, grp, 0)

    def drain_out(i):
        f, d = fd(i)
        pltpu.make_async_copy(out_hbm.at[f, d], out_v, semo).wait()

    fire_chunk(0, 0)
    fire_chunk(0, 1)
    load_x(fd(0)[0])

    for i in range(_SPW):
        f, d = fd(i)
        if i > 0:
            drain_out(i - 1)
            pl.when(d == 0)(lambda f=f: load_x(f))
        wait_chunk(i, 0)
        stripe_pass(i, 0)
        fire_chunk(i, 2)
        fire_tail(i)
        wait_chunk(i, 1)
        stripe_pass(i, 1)
        if i + 1 < _SPW:
            fire_chunk(i + 1, 0)
        wait_chunk(i, 2)
        wait_tail(i)
        stripe_pass(i, 2)
        if i + 1 < _SPW:
            fire_chunk(i + 1, 1)
        pltpu.async_copy(out_v, out_hbm.at[f, d], semo)

    drain_out(_SPW - 1)


def kernel(x, tables):
    tab_t = tables.transpose(0, 2, 1)
    pad = 128 - (_VOCAB - _VTAIL)
    tail = jnp.pad(tab_t[:, :, _VTAIL:], ((0, 0), (0, 0), (0, pad)))
    out_t = _gather_kernel(x.T, tab_t, tail)
    return out_t.transpose(0, 2, 1)


# speedup vs baseline: 2.6764x; 2.6764x over previous
import functools

import jax
import jax.numpy as jnp
from jax import lax
from jax.experimental import pallas as pl
from jax.experimental.pallas import tpu as pltpu
from jax.experimental.pallas import tpu_sc as plsc

_N_FIELDS = 26
_VOCAB = 100000
_DIM = 16
_BATCH = 16384

_NC, _NS, _L = 2, 16, 16
_NW = _NC * _NS
_NSTRIPE = _N_FIELDS * _DIM
_SPW = _NSTRIPE // _NW
_NH = 4
_HB = _BATCH // _NH

_mesh = plsc.VectorSubcoreMesh(
    core_axis_name="c", subcore_axis_name="s", num_cores=_NC, num_subcores=_NS
)


@functools.partial(
    pl.kernel,
    out_type=jax.ShapeDtypeStruct((_N_FIELDS, _DIM, _BATCH), jnp.float32),
    mesh=_mesh,
    compiler_params=pltpu.CompilerParams(
        needs_layout_passes=False, use_tc_tiling_on_sc=True
    ),
    scratch_types=[
        pltpu.VMEM((_VOCAB,), jnp.float32),
        pltpu.VMEM((_BATCH,), jnp.int32),
        pltpu.VMEM((2 * _HB,), jnp.float32),
        pltpu.SemaphoreType.DMA,
        pltpu.SemaphoreType.DMA,
        pltpu.SemaphoreType.DMA,
    ],
)
def _gather_kernel(xt_hbm, tab_hbm, out_hbm, stripe_v, x_v, out_v, sem0, sem1, sems_t):
    wid = lax.axis_index("s") * _NC + lax.axis_index("c")
    s0 = wid * _SPW
    sems = (sem0, sem1)

    def do_stripe(i, carry):
        s = s0 + i
        f = s // _DIM
        d = s % _DIM

        pltpu.async_copy(tab_hbm.at[f, d], stripe_v, sems_t)

        @pl.when(jnp.logical_or(i == 0, d == 0))
        def _load_x():
            pltpu.sync_copy(xt_hbm.at[f], x_v)

        pltpu.make_async_copy(tab_hbm.at[f, d], stripe_v, sems_t).wait()

        def do_quarter(h):
            slot = h % 2
            base = h * _HB
            ob = out_v.at[pl.ds(slot * _HB, _HB)]

            def _drain_prev():
                pltpu.make_async_copy(
                    out_hbm.at[f, d, pl.ds(base, _HB)], ob, sems[slot]
                ).wait()

            if h >= 2:
                _drain_prev()
            else:
                pl.when(i > 0)(_drain_prev)

            def grp(g, c):
                for k in range(16):
                    off = g * (16 * _L) + k * _L
                    xv = x_v[pl.ds(base + off, _L)]
                    out_v[pl.ds(slot * _HB + off, _L)] = plsc.load_gather(
                        stripe_v, [xv]
                    )
                return c

            lax.fori_loop(0, _HB // (16 * _L), grp, 0)
            pltpu.async_copy(ob, out_hbm.at[f, d, pl.ds(base, _HB)], sems[slot])

        for h in range(_NH):
            do_quarter(h)
        return carry

    lax.fori_loop(0, _SPW, do_stripe, 0)

    last = s0 + _SPW - 1
    lf = last // _DIM
    ld = last % _DIM
    for h in range(2):
        pltpu.make_async_copy(
            out_hbm.at[lf, ld, pl.ds((2 + h) * _HB, _HB)],
            out_v.at[pl.ds(h * _HB, _HB)],
            sems[h],
        ).wait()


def kernel(x, tables):
    tab_t = tables.transpose(0, 2, 1)
    out_t = _gather_kernel(x.T, tab_t)
    return out_t.transpose(0, 2, 1)


# speedup vs baseline: 2.6970x; 1.0077x over previous
import functools

import jax
import jax.numpy as jnp
from jax import lax
from jax.experimental import pallas as pl
from jax.experimental.pallas import tpu as pltpu
from jax.experimental.pallas import tpu_sc as plsc

_N_FIELDS = 26
_VOCAB = 100000
_DIM = 16
_BATCH = 16384

_NC, _NS, _L = 2, 16, 16
_NW = _NC * _NS
_NSTRIPE = _N_FIELDS * _DIM
_SPW = _NSTRIPE // _NW
_NH = 4
_HB = _BATCH // _NH

_mesh = plsc.VectorSubcoreMesh(
    core_axis_name="c", subcore_axis_name="s", num_cores=_NC, num_subcores=_NS
)


@functools.partial(
    pl.kernel,
    out_type=jax.ShapeDtypeStruct((_N_FIELDS, _DIM, _BATCH), jnp.float32),
    mesh=_mesh,
    compiler_params=pltpu.CompilerParams(
        needs_layout_passes=False, use_tc_tiling_on_sc=True
    ),
    scratch_types=[
        pltpu.VMEM((_VOCAB,), jnp.float32),
        pltpu.VMEM((_BATCH,), jnp.int32),
        pltpu.VMEM((2 * _HB,), jnp.float32),
        pltpu.SemaphoreType.DMA,
        pltpu.SemaphoreType.DMA,
    ],
)
def _gather_kernel(xt_hbm, tab_hbm, out_hbm, stripe_v, x_v, out_v, sem0, sem1):
    wid = lax.axis_index("s") * _NC + lax.axis_index("c")
    s0 = wid * _SPW
    sems = (sem0, sem1)

    def do_stripe(i, carry):
        s = s0 + i
        f = s // _DIM
        d = s % _DIM

        @pl.when(jnp.logical_or(i == 0, d == 0))
        def _load_x():
            pltpu.sync_copy(xt_hbm.at[f], x_v)

        pltpu.sync_copy(tab_hbm.at[f, d], stripe_v)

        def do_quarter(h):
            slot = h % 2
            base = h * _HB
            ob = out_v.at[pl.ds(slot * _HB, _HB)]

            def _drain_prev():
                pltpu.make_async_copy(
                    out_hbm.at[f, d, pl.ds(base, _HB)], ob, sems[slot]
                ).wait()

            if h >= 2:
                _drain_prev()
            else:
                pl.when(i > 0)(_drain_prev)

            def grp(g, c):
                for k in range(16):
                    off = g * (16 * _L) + k * _L
                    xv = x_v[pl.ds(base + off, _L)]
                    out_v[pl.ds(slot * _HB + off, _L)] = plsc.load_gather(
                        stripe_v, [xv]
                    )
                return c

            lax.fori_loop(0, _HB // (16 * _L), grp, 0)
            pltpu.async_copy(ob, out_hbm.at[f, d, pl.ds(base, _HB)], sems[slot])

        for h in range(_NH):
            do_quarter(h)
        return carry

    lax.fori_loop(0, _SPW, do_stripe, 0)

    last = s0 + _SPW - 1
    lf = last // _DIM
    ld = last % _DIM
    for h in range(2):
        pltpu.make_async_copy(
            out_hbm.at[lf, ld, pl.ds((2 + h) * _HB, _HB)],
            out_v.at[pl.ds(h * _HB, _HB)],
            sems[h],
        ).wait()


def kernel(x, tables):
    tab_t = tables.transpose(0, 2, 1)
    out_t = _gather_kernel(x.T, tab_t)
    return out_t.transpose(0, 2, 1)
